# Initial kernel scaffold; baseline (speedup 1.0000x reference)
import jax
import jax.numpy as jnp
from jax.experimental import pallas as pl


def kernel(x, edge_index, edge_weight, Ws, bias):
    raise NotImplementedError("write your pallas kernel here")


# speedup vs baseline: 8.2475x; 8.2475x over previous
import jax
import jax.numpy as jnp
from jax import lax
from jax.experimental import pallas as pl
from jax.experimental.pallas import tpu as pltpu
from jax.experimental.pallas import tpu_sc as plsc

N = 10000
NPAD = 10240
E = 320000
D = 128
DH = D // 2
NC, NS, LANE = 2, 16, 16
NW = NC * NS
EB = 80

EPT_D = E // NW
NB_D = EPT_D // EB

EPT_S = E // NS
NB_S = EPT_S // EB

RPT = NPAD // NS
ZBR = 128


def _mesh():
    return plsc.VectorSubcoreMesh(core_axis_name="c", subcore_axis_name="s")


def _deg_body(src3, dst3, ew3, degp, srcv, dstv, ewv, wv, zf, deg_sh):
    c = lax.axis_index("c")
    s = lax.axis_index("s")
    wid = c * NS + s

    @pl.when(s == 0)
    def _():
        @pl.loop(0, N // LANE)
        def _(i):
            zf[pl.ds(i * LANE, LANE)] = jnp.zeros((LANE,), jnp.float32)

        pltpu.sync_copy(zf, deg_sh)

    pltpu.sync_copy(src3.at[wid], srcv)
    pltpu.sync_copy(dst3.at[wid], dstv)
    pltpu.sync_copy(ew3.at[wid], ewv)
    plsc.subcore_barrier()

    @pl.loop(0, NB_D)
    def _(b):
        for i in range(EB // LANE):
            sl = pl.ds(i * LANE, LANE)
            s16 = srcv[b, sl]
            wv[sl] = jnp.where(s16 == dstv[b, sl], 0.0, ewv[b, sl])

        pltpu.sync_copy(wv, deg_sh.at[srcv.at[b]], add=True)

    plsc.subcore_barrier()

    @pl.when(s == 0)
    def _():
        pltpu.sync_copy(deg_sh, zf)
        pltpu.sync_copy(zf, degp.at[pl.ds(c * N, N)])


def _deg_call(src3, dst3, ew3):
    return pl.kernel(
        _deg_body,
        out_type=jax.ShapeDtypeStruct((NC * N,), jnp.float32),
        mesh=_mesh(),
        scratch_types=[
            pltpu.VMEM((NB_D, EB), jnp.int32),
            pltpu.VMEM((NB_D, EB), jnp.int32),
            pltpu.VMEM((NB_D, EB), jnp.float32),
            pltpu.VMEM((EB,), jnp.float32),
            pltpu.VMEM((N,), jnp.float32),
            pltpu.VMEM_SHARED((N,), jnp.float32),
        ],
    )(src3, dst3, ew3)


def _dinv_body(degp_ref, dinv_ref):
    d = degp_ref[0, :] + degp_ref[1, :]
    safe = jnp.where(d > 0.0, d, 1.0)
    dinv_ref[:] = jnp.where(d > 0.0, lax.rsqrt(safe), 0.0)


def _dinv_call(degp):
    return pl.pallas_call(
        _dinv_body,
        out_shape=jax.ShapeDtypeStruct((N,), jnp.float32),
    )(degp)


def _scale_rows(rows, n16c, i):
    for l in range(LANE):
        n16 = jnp.full((LANE,), n16c[l], jnp.float32)
        r = i * LANE + l
        for j in range(DH // LANE):
            slj = pl.ds(j * LANE, LANE)
            rows[r, slj] = rows[r, slj] * n16


def _zero_acc(zb, acc_sh, s):
    @pl.loop(0, ZBR)
    def _(r):
        for j in range(DH // LANE):
            zb[r, pl.ds(j * LANE, LANE)] = jnp.zeros((LANE,), jnp.float32)

    for t in range(RPT // ZBR):
        pltpu.sync_copy(zb, acc_sh.at[pl.ds(s * RPT + t * ZBR, ZBR)])


def _dump_acc(zb, acc_sh, part, c, s):
    for t in range(RPT // ZBR):
        off = s * RPT + t * ZBR
        pltpu.sync_copy(acc_sh.at[pl.ds(off, ZBR)], zb)
        pltpu.sync_copy(zb, part.at[c, pl.ds(off, ZBR)])


def _gather_half(xl_h, xr_h, idx, rows, c):
    @pl.when(c == 0)
    def _():
        pltpu.sync_copy(xl_h.at[idx], rows)

    @pl.when(c == 1)
    def _():
        pltpu.sync_copy(xr_h.at[idx], rows)


def _spmm_body(src3, dst3, ew3, dinv_h, yl_h, yr_h, part,
               srcv, dstv, ewv, dsb, ddb, dvb, rows, zb, acc_sh, dv_sh):
    c = lax.axis_index("c")
    s = lax.axis_index("s")

    _zero_acc(zb, acc_sh, s)

    @pl.when(s == 0)
    def _():
        pltpu.sync_copy(dinv_h, dvb)
        pltpu.sync_copy(dvb, dv_sh)

    pltpu.sync_copy(src3.at[s], srcv)
    pltpu.sync_copy(dst3.at[s], dstv)
    pltpu.sync_copy(ew3.at[s], ewv)
    plsc.subcore_barrier()

    @pl.loop(0, NB_S)
    def _(b):
        _gather_half(yl_h, yr_h, srcv.at[b], rows, c)
        pltpu.sync_copy(dv_sh.at[srcv.at[b]], dsb)
        pltpu.sync_copy(dv_sh.at[dstv.at[b]], ddb)
        for i in range(EB // LANE):
            sl = pl.ds(i * LANE, LANE)
            w16 = jnp.where(srcv[b, sl] == dstv[b, sl], 0.0, ewv[b, sl])
            n16c = -(dsb[sl] * w16 * ddb[sl])
            _scale_rows(rows, n16c, i)

        pltpu.sync_copy(rows, acc_sh.at[dstv.at[b]], add=True)

    plsc.subcore_barrier()
    _dump_acc(zb, acc_sh, part, c, s)


def _spmm_call(src3, dst3, ew3, dinv, yl, yr):
    return pl.kernel(
        _spmm_body,
        out_type=jax.ShapeDtypeStruct((NC, NPAD, DH), jnp.float32),
        mesh=_mesh(),
        compiler_params=pltpu.CompilerParams(use_tc_tiling_on_sc=False),
        scratch_types=[
            pltpu.VMEM((NB_S, EB), jnp.int32),
            pltpu.VMEM((NB_S, EB), jnp.int32),
            pltpu.VMEM((NB_S, EB), jnp.float32),
            pltpu.VMEM((EB,), jnp.float32),
            pltpu.VMEM((EB,), jnp.float32),
            pltpu.VMEM((N,), jnp.float32),
            pltpu.VMEM((EB, DH), jnp.float32),
            pltpu.VMEM((ZBR, DH), jnp.float32),
            pltpu.VMEM_SHARED((NPAD, DH), jnp.float32),
            pltpu.VMEM_SHARED((N,), jnp.float32),
        ],
    )(src3, dst3, ew3, dinv, yl, yr)


def _post_body(x_ref, t1_ref, q_ref, ws_ref, b_ref, o_ref):
    xb = x_ref[...]
    t1 = jnp.concatenate([t1_ref[0], t1_ref[1]], axis=1)
    t2 = 2.0 * jnp.concatenate([q_ref[0], q_ref[1]], axis=1) - xb
    dn = (((1,), (1,)), ((), ()))
    acc = lax.dot_general(xb, ws_ref[0], dn, preferred_element_type=jnp.float32)
    acc = acc + lax.dot_general(t1, ws_ref[1], dn,
                                preferred_element_type=jnp.float32)
    acc = acc + lax.dot_general(t2, ws_ref[2], dn,
                                preferred_element_type=jnp.float32)
    o_ref[...] = acc + b_ref[...]


def _post_call(x, part1, part2, Ws, bias2):
    nblk = 5
    br = N // nblk
    return pl.pallas_call(
        _post_body,
        grid=(nblk,),
        in_specs=[
            pl.BlockSpec((br, D), lambda i: (i, 0)),
            pl.BlockSpec((NC, br, DH), lambda i: (0, i, 0)),
            pl.BlockSpec((NC, br, DH), lambda i: (0, i, 0)),
            pl.BlockSpec((3, D, D), lambda i: (0, 0, 0)),
            pl.BlockSpec((1, D), lambda i: (0, 0)),
        ],
        out_specs=pl.BlockSpec((br, D), lambda i: (i, 0)),
        out_shape=jax.ShapeDtypeStruct((N, D), jnp.float32),
    )(x, part1, part2, Ws, bias2)


def kernel(x, edge_index, edge_weight, Ws, bias):
    src_d = edge_index[0].reshape(NW, NB_D, EB)
    dst_d = edge_index[1].reshape(NW, NB_D, EB)
    ew_d = edge_weight.reshape(NW, NB_D, EB)
    src_s = edge_index[0].reshape(NS, NB_S, EB)
    dst_s = edge_index[1].reshape(NS, NB_S, EB)
    ew_s = edge_weight.reshape(NS, NB_S, EB)
    xl = x[:, :DH]
    xr = x[:, DH:]
    degp = _deg_call(src_d, dst_d, ew_d).reshape(NC, N)
    dinv = _dinv_call(degp)
    part1 = _spmm_call(src_s, dst_s, ew_s, dinv, xl, xr)
    part2 = _spmm_call(src_s, dst_s, ew_s, dinv, part1[0], part1[1])
    return _post_call(x, part1, part2, Ws, bias.reshape(1, D))


# speedup vs baseline: 9.4060x; 1.1405x over previous
import jax
import jax.numpy as jnp
from jax import lax
from jax.experimental import pallas as pl
from jax.experimental.pallas import tpu as pltpu
from jax.experimental.pallas import tpu_sc as plsc

N = 10000
NPAD = 10240
E = 320000
D = 128
DH = D // 2
NC, NS, LANE = 2, 16, 16
NW = NC * NS
EB = 80

EPT_D = E // NW
NB_D = EPT_D // EB

EPT_S = E // NS
NB_S = EPT_S // EB

RPT = NPAD // NS
ZBR = 128


def _mesh():
    return plsc.VectorSubcoreMesh(core_axis_name="c", subcore_axis_name="s")


def _deg_body(src3, dst3, ew3, degp, srcv, dstv, ewv, wv, zf, deg_sh):
    c = lax.axis_index("c")
    s = lax.axis_index("s")
    wid = c * NS + s

    @pl.when(s == 0)
    def _():
        @pl.loop(0, N // LANE)
        def _(i):
            zf[pl.ds(i * LANE, LANE)] = jnp.zeros((LANE,), jnp.float32)

        pltpu.sync_copy(zf, deg_sh)

    pltpu.sync_copy(src3.at[wid], srcv)
    pltpu.sync_copy(dst3.at[wid], dstv)
    pltpu.sync_copy(ew3.at[wid], ewv)
    plsc.subcore_barrier()

    @pl.loop(0, NB_D)
    def _(b):
        for i in range(EB // LANE):
            sl = pl.ds(i * LANE, LANE)
            s16 = srcv[b, sl]
            wv[sl] = jnp.where(s16 == dstv[b, sl], 0.0, ewv[b, sl])

        pltpu.sync_copy(wv, deg_sh.at[srcv.at[b]], add=True)

    plsc.subcore_barrier()

    @pl.when(s == 0)
    def _():
        pltpu.sync_copy(deg_sh, zf)
        pltpu.sync_copy(zf, degp.at[pl.ds(c * N, N)])


def _deg_call(src3, dst3, ew3):
    return pl.kernel(
        _deg_body,
        out_type=jax.ShapeDtypeStruct((NC * N,), jnp.float32),
        mesh=_mesh(),
        scratch_types=[
            pltpu.VMEM((NB_D, EB), jnp.int32),
            pltpu.VMEM((NB_D, EB), jnp.int32),
            pltpu.VMEM((NB_D, EB), jnp.float32),
            pltpu.VMEM((EB,), jnp.float32),
            pltpu.VMEM((N,), jnp.float32),
            pltpu.VMEM_SHARED((N,), jnp.float32),
        ],
    )(src3, dst3, ew3)


def _dinv_body(degp_ref, x_ref, dinv_ref, xl_ref, xr_ref):
    d = degp_ref[0, :] + degp_ref[1, :]
    safe = jnp.where(d > 0.0, d, 1.0)
    dv = jnp.reshape(jnp.where(d > 0.0, lax.rsqrt(safe), 0.0), (N, 1))
    dinv_ref[...] = dv
    x2 = dv * x_ref[...]
    xl_ref[...] = x2[:, :DH]
    xr_ref[...] = x2[:, DH:]


def _dinv_call(degp, x):
    return pl.pallas_call(
        _dinv_body,
        out_shape=(
            jax.ShapeDtypeStruct((N, 1), jnp.float32),
            jax.ShapeDtypeStruct((N, DH), jnp.float32),
            jax.ShapeDtypeStruct((N, DH), jnp.float32),
        ),
    )(degp, x)


def _mid_body(q_ref, dinv_ref, t1_ref, yl_ref, yr_ref):
    dv = dinv_ref[...]
    ql = -dv * q_ref[0]
    qr = -dv * q_ref[1]
    t1_ref[...] = jnp.concatenate([ql, qr], axis=1)
    yl_ref[...] = dv * ql
    yr_ref[...] = dv * qr


def _mid_call(part1, dinv):
    nblk = 5
    br = N // nblk
    return pl.pallas_call(
        _mid_body,
        grid=(nblk,),
        in_specs=[
            pl.BlockSpec((NC, br, DH), lambda i: (0, i, 0)),
            pl.BlockSpec((br, 1), lambda i: (i, 0)),
        ],
        out_specs=(
            pl.BlockSpec((br, D), lambda i: (i, 0)),
            pl.BlockSpec((br, DH), lambda i: (i, 0)),
            pl.BlockSpec((br, DH), lambda i: (i, 0)),
        ),
        out_shape=(
            jax.ShapeDtypeStruct((N, D), jnp.float32),
            jax.ShapeDtypeStruct((N, DH), jnp.float32),
            jax.ShapeDtypeStruct((N, DH), jnp.float32),
        ),
    )(part1, dinv)


def _scale_rows(rows, n16c, i):
    for l in range(LANE):
        n16 = jnp.full((LANE,), n16c[l], jnp.float32)
        r = i * LANE + l
        for j in range(DH // LANE):
            slj = pl.ds(j * LANE, LANE)
            rows[r, slj] = rows[r, slj] * n16


def _zero_acc(zb, acc_sh, s):
    @pl.loop(0, ZBR)
    def _(r):
        for j in range(DH // LANE):
            zb[r, pl.ds(j * LANE, LANE)] = jnp.zeros((LANE,), jnp.float32)

    for t in range(RPT // ZBR):
        pltpu.sync_copy(zb, acc_sh.at[pl.ds(s * RPT + t * ZBR, ZBR)])


def _dump_acc(zb, acc_sh, part, c, s):
    for t in range(RPT // ZBR):
        off = s * RPT + t * ZBR
        pltpu.sync_copy(acc_sh.at[pl.ds(off, ZBR)], zb)
        pltpu.sync_copy(zb, part.at[c, pl.ds(off, ZBR)])


def _gather_half(xl_h, xr_h, idx, rows, c):
    @pl.when(c == 0)
    def _():
        pltpu.sync_copy(xl_h.at[idx], rows)

    @pl.when(c == 1)
    def _():
        pltpu.sync_copy(xr_h.at[idx], rows)


def _spmm_body(src3, dst3, ew3, yl_h, yr_h, part,
               srcv, dstv, ewv, rows, zb, acc_sh):
    c = lax.axis_index("c")
    s = lax.axis_index("s")

    _zero_acc(zb, acc_sh, s)
    pltpu.sync_copy(src3.at[s], srcv)
    pltpu.sync_copy(dst3.at[s], dstv)
    pltpu.sync_copy(ew3.at[s], ewv)
    plsc.subcore_barrier()

    @pl.loop(0, NB_S)
    def _(b):
        _gather_half(yl_h, yr_h, srcv.at[b], rows, c)
        for i in range(EB // LANE):
            sl = pl.ds(i * LANE, LANE)
            n16c = jnp.where(srcv[b, sl] == dstv[b, sl], 0.0, ewv[b, sl])
            _scale_rows(rows, n16c, i)

        pltpu.sync_copy(rows, acc_sh.at[dstv.at[b]], add=True)

    plsc.subcore_barrier()
    _dump_acc(zb, acc_sh, part, c, s)


def _spmm_call(src3, dst3, ew3, yl, yr):
    return pl.kernel(
        _spmm_body,
        out_type=jax.ShapeDtypeStruct((NC, NPAD, DH), jnp.float32),
        mesh=_mesh(),
        compiler_params=pltpu.CompilerParams(use_tc_tiling_on_sc=False),
        scratch_types=[
            pltpu.VMEM((NB_S, EB), jnp.int32),
            pltpu.VMEM((NB_S, EB), jnp.int32),
            pltpu.VMEM((NB_S, EB), jnp.float32),
            pltpu.VMEM((EB, DH), jnp.float32),
            pltpu.VMEM((ZBR, DH), jnp.float32),
            pltpu.VMEM_SHARED((NPAD, DH), jnp.float32),
        ],
    )(src3, dst3, ew3, yl, yr)


def _post_body(x_ref, t1_ref, q_ref, dinv_ref, ws_ref, b_ref, o_ref):
    xb = x_ref[...]
    t1 = t1_ref[...]
    dv = dinv_ref[...]
    t2 = -2.0 * dv * jnp.concatenate([q_ref[0], q_ref[1]], axis=1) - xb
    dn = (((1,), (1,)), ((), ()))
    acc = lax.dot_general(xb, ws_ref[0], dn, preferred_element_type=jnp.float32)
    acc = acc + lax.dot_general(t1, ws_ref[1], dn,
                                preferred_element_type=jnp.float32)
    acc = acc + lax.dot_general(t2, ws_ref[2], dn,
                                preferred_element_type=jnp.float32)
    o_ref[...] = acc + b_ref[...]


def _post_call(x, t1, part2, dinv, Ws, bias2):
    nblk = 5
    br = N // nblk
    return pl.pallas_call(
        _post_body,
        grid=(nblk,),
        in_specs=[
            pl.BlockSpec((br, D), lambda i: (i, 0)),
            pl.BlockSpec((br, D), lambda i: (i, 0)),
            pl.BlockSpec((NC, br, DH), lambda i: (0, i, 0)),
            pl.BlockSpec((br, 1), lambda i: (i, 0)),
            pl.BlockSpec((3, D, D), lambda i: (0, 0, 0)),
            pl.BlockSpec((1, D), lambda i: (0, 0)),
        ],
        out_specs=pl.BlockSpec((br, D), lambda i: (i, 0)),
        out_shape=jax.ShapeDtypeStruct((N, D), jnp.float32),
    )(x, t1, part2, dinv, Ws, bias2)


def kernel(x, edge_index, edge_weight, Ws, bias):
    src_d = edge_index[0].reshape(NW, NB_D, EB)
    dst_d = edge_index[1].reshape(NW, NB_D, EB)
    ew_d = edge_weight.reshape(NW, NB_D, EB)
    src_s = edge_index[0].reshape(NS, NB_S, EB)
    dst_s = edge_index[1].reshape(NS, NB_S, EB)
    ew_s = edge_weight.reshape(NS, NB_S, EB)
    degp = _deg_call(src_d, dst_d, ew_d).reshape(NC, N)
    dinv, xl, xr = _dinv_call(degp, x)
    part1 = _spmm_call(src_s, dst_s, ew_s, xl, xr)
    t1, yl, yr = _mid_call(part1[:, :N], dinv)
    part2 = _spmm_call(src_s, dst_s, ew_s, yl, yr)
    return _post_call(x, t1, part2, dinv, Ws, bias.reshape(1, D))


# speedup vs baseline: 10.1097x; 1.0748x over previous
import jax
import jax.numpy as jnp
from jax import lax
from jax.experimental import pallas as pl
from jax.experimental.pallas import tpu as pltpu
from jax.experimental.pallas import tpu_sc as plsc

N = 10000
NPAD = 10240
E = 320000
D = 128
DH = D // 2
NC, NS, LANE = 2, 16, 16
NW = NC * NS
EB = 80

EPT_D = E // NW
NB_D = EPT_D // EB

EBS = 128
NB_S = 157
EPT_S = NB_S * EBS
EPAD = NS * EPT_S

RPT = NPAD // NS
ZBR = 128


def _mesh():
    return plsc.VectorSubcoreMesh(core_axis_name="c", subcore_axis_name="s")


def _deg_body(src3, dst3, ew3, degp, srcv, dstv, ewv, wv, zf, deg_sh):
    c = lax.axis_index("c")
    s = lax.axis_index("s")
    wid = c * NS + s

    @pl.when(s == 0)
    def _():
        @pl.loop(0, N // LANE)
        def _(i):
            zf[pl.ds(i * LANE, LANE)] = jnp.zeros((LANE,), jnp.float32)

        pltpu.sync_copy(zf, deg_sh)

    pltpu.sync_copy(src3.at[wid], srcv)
    pltpu.sync_copy(dst3.at[wid], dstv)
    pltpu.sync_copy(ew3.at[wid], ewv)
    plsc.subcore_barrier()

    @pl.loop(0, NB_D)
    def _(b):
        for i in range(EB // LANE):
            sl = pl.ds(i * LANE, LANE)
            s16 = srcv[b, sl]
            wv[sl] = jnp.where(s16 == dstv[b, sl], 0.0, ewv[b, sl])

        pltpu.sync_copy(wv, deg_sh.at[srcv.at[b]], add=True)

    plsc.subcore_barrier()

    @pl.when(s == 0)
    def _():
        pltpu.sync_copy(deg_sh, zf)
        pltpu.sync_copy(zf, degp.at[pl.ds(c * N, N)])


def _deg_call(src3, dst3, ew3):
    return pl.kernel(
        _deg_body,
        out_type=jax.ShapeDtypeStruct((NC * N,), jnp.float32),
        mesh=_mesh(),
        scratch_types=[
            pltpu.VMEM((NB_D, EB), jnp.int32),
            pltpu.VMEM((NB_D, EB), jnp.int32),
            pltpu.VMEM((NB_D, EB), jnp.float32),
            pltpu.VMEM((EB,), jnp.float32),
            pltpu.VMEM((N,), jnp.float32),
            pltpu.VMEM_SHARED((N,), jnp.float32),
        ],
    )(src3, dst3, ew3)


def _dinv_body(degp_ref, x_ref, dinv_ref, xl_ref, xr_ref):
    d = degp_ref[0, :] + degp_ref[1, :]
    safe = jnp.where(d > 0.0, d, 1.0)
    dv = jnp.reshape(jnp.where(d > 0.0, lax.rsqrt(safe), 0.0), (N, 1))
    dinv_ref[...] = dv
    x2 = dv * x_ref[...]
    xl_ref[...] = x2[:, :DH]
    xr_ref[...] = x2[:, DH:]


def _dinv_call(degp, x):
    return pl.pallas_call(
        _dinv_body,
        out_shape=(
            jax.ShapeDtypeStruct((N, 1), jnp.float32),
            jax.ShapeDtypeStruct((N, DH), jnp.float32),
            jax.ShapeDtypeStruct((N, DH), jnp.float32),
        ),
    )(degp, x)


def _mid_body(q_ref, dinv_ref, t1_ref, yl_ref, yr_ref):
    dv = dinv_ref[...]
    ql = -dv * q_ref[0]
    qr = -dv * q_ref[1]
    t1_ref[...] = jnp.concatenate([ql, qr], axis=1)
    yl_ref[...] = dv * ql
    yr_ref[...] = dv * qr


def _mid_call(part1, dinv):
    nblk = 5
    br = N // nblk
    return pl.pallas_call(
        _mid_body,
        grid=(nblk,),
        in_specs=[
            pl.BlockSpec((NC, br, DH), lambda i: (0, i, 0)),
            pl.BlockSpec((br, 1), lambda i: (i, 0)),
        ],
        out_specs=(
            pl.BlockSpec((br, D), lambda i: (i, 0)),
            pl.BlockSpec((br, DH), lambda i: (i, 0)),
            pl.BlockSpec((br, DH), lambda i: (i, 0)),
        ),
        out_shape=(
            jax.ShapeDtypeStruct((N, D), jnp.float32),
            jax.ShapeDtypeStruct((N, DH), jnp.float32),
            jax.ShapeDtypeStruct((N, DH), jnp.float32),
        ),
    )(part1, dinv)


def _scale_rows(rows, n16c, i):
    for l in range(LANE):
        n16 = jnp.full((LANE,), n16c[l], jnp.float32)
        r = i * LANE + l
        for j in range(DH // LANE):
            slj = pl.ds(j * LANE, LANE)
            rows[r, slj] = rows[r, slj] * n16


def _zero_acc(zb, acc_sh, s):
    @pl.loop(0, ZBR)
    def _(r):
        for j in range(DH // LANE):
            zb[r, pl.ds(j * LANE, LANE)] = jnp.zeros((LANE,), jnp.float32)

    for t in range(RPT // ZBR):
        pltpu.sync_copy(zb, acc_sh.at[pl.ds(s * RPT + t * ZBR, ZBR)])


def _dump_acc(zb, acc_sh, part, c, s):
    for t in range(RPT // ZBR):
        off = s * RPT + t * ZBR
        pltpu.sync_copy(acc_sh.at[pl.ds(off, ZBR)], zb)
        pltpu.sync_copy(zb, part.at[c, pl.ds(off, ZBR)])


def _gather_half(xl_h, xr_h, idx, rows, c):
    @pl.when(c == 0)
    def _():
        pltpu.sync_copy(xl_h.at[idx], rows)

    @pl.when(c == 1)
    def _():
        pltpu.sync_copy(xr_h.at[idx], rows)


def _spmm_body(src3, dst3, ew3, yl_h, yr_h, part,
               srcv, dstv, ewv, rows, zb, acc_sh):
    c = lax.axis_index("c")
    s = lax.axis_index("s")

    _zero_acc(zb, acc_sh, s)
    pltpu.sync_copy(src3.at[s], srcv)
    pltpu.sync_copy(dst3.at[s], dstv)
    pltpu.sync_copy(ew3.at[s], ewv)
    plsc.subcore_barrier()

    @pl.loop(0, NB_S)
    def _(b):
        _gather_half(yl_h, yr_h, srcv.at[b], rows, c)
        for i in range(EBS // LANE):
            sl = pl.ds(i * LANE, LANE)
            n16c = jnp.where(srcv[b, sl] == dstv[b, sl], 0.0, ewv[b, sl])
            _scale_rows(rows, n16c, i)

        pltpu.sync_copy(rows, acc_sh.at[dstv.at[b]], add=True)

    plsc.subcore_barrier()
    _dump_acc(zb, acc_sh, part, c, s)


def _spmm_call(src3, dst3, ew3, yl, yr):
    return pl.kernel(
        _spmm_body,
        out_type=jax.ShapeDtypeStruct((NC, NPAD, DH), jnp.float32),
        mesh=_mesh(),
        compiler_params=pltpu.CompilerParams(use_tc_tiling_on_sc=False),
        scratch_types=[
            pltpu.VMEM((NB_S, EBS), jnp.int32),
            pltpu.VMEM((NB_S, EBS), jnp.int32),
            pltpu.VMEM((NB_S, EBS), jnp.float32),
            pltpu.VMEM((EBS, DH), jnp.float32),
            pltpu.VMEM((ZBR, DH), jnp.float32),
            pltpu.VMEM_SHARED((NPAD, DH), jnp.float32),
        ],
    )(src3, dst3, ew3, yl, yr)


def _post_body(x_ref, t1_ref, q_ref, dinv_ref, ws_ref, b_ref, o_ref):
    xb = x_ref[...]
    t1 = t1_ref[...]
    dv = dinv_ref[...]
    t2 = -2.0 * dv * jnp.concatenate([q_ref[0], q_ref[1]], axis=1) - xb
    dn = (((1,), (1,)), ((), ()))
    acc = lax.dot_general(xb, ws_ref[0], dn, preferred_element_type=jnp.float32)
    acc = acc + lax.dot_general(t1, ws_ref[1], dn,
                                preferred_element_type=jnp.float32)
    acc = acc + lax.dot_general(t2, ws_ref[2], dn,
                                preferred_element_type=jnp.float32)
    o_ref[...] = acc + b_ref[...]


def _post_call(x, t1, part2, dinv, Ws, bias2):
    nblk = 5
    br = N // nblk
    return pl.pallas_call(
        _post_body,
        grid=(nblk,),
        in_specs=[
            pl.BlockSpec((br, D), lambda i: (i, 0)),
            pl.BlockSpec((br, D), lambda i: (i, 0)),
            pl.BlockSpec((NC, br, DH), lambda i: (0, i, 0)),
            pl.BlockSpec((br, 1), lambda i: (i, 0)),
            pl.BlockSpec((3, D, D), lambda i: (0, 0, 0)),
            pl.BlockSpec((1, D), lambda i: (0, 0)),
        ],
        out_specs=pl.BlockSpec((br, D), lambda i: (i, 0)),
        out_shape=jax.ShapeDtypeStruct((N, D), jnp.float32),
    )(x, t1, part2, dinv, Ws, bias2)


def kernel(x, edge_index, edge_weight, Ws, bias):
    src_d = edge_index[0].reshape(NW, NB_D, EB)
    dst_d = edge_index[1].reshape(NW, NB_D, EB)
    ew_d = edge_weight.reshape(NW, NB_D, EB)
    padn = EPAD - E
    src_s = jnp.concatenate(
        [edge_index[0], jnp.zeros((padn,), jnp.int32)]).reshape(NS, NB_S, EBS)
    dst_s = jnp.concatenate(
        [edge_index[1], jnp.zeros((padn,), jnp.int32)]).reshape(NS, NB_S, EBS)
    ew_s = jnp.concatenate(
        [edge_weight, jnp.zeros((padn,), jnp.float32)]).reshape(NS, NB_S, EBS)
    degp = _deg_call(src_d, dst_d, ew_d).reshape(NC, N)
    dinv, xl, xr = _dinv_call(degp, x)
    part1 = _spmm_call(src_s, dst_s, ew_s, xl, xr)
    t1, yl, yr = _mid_call(part1[:, :N], dinv)
    part2 = _spmm_call(src_s, dst_s, ew_s, yl, yr)
    return _post_call(x, t1, part2, dinv, Ws, bias.reshape(1, D))
